# baseline (speedup 1.0000x reference)
import functools

import jax
import jax.numpy as jnp
from jax.experimental import pallas as pl

N_TOK = 8192
N_EXP = 64
K = 8
BLOCK = 1024


def _gate_kernel(x_ref, w_ref, out_w_ref, out_i_ref):
    x = x_ref[...]
    w = w_ref[...]
    logits = jax.lax.dot_general(
        x, w, (((1,), (1,)), ((), ())), preferred_element_type=jnp.float32
    )
    m = jnp.max(logits, axis=1, keepdims=True)
    s = jnp.sum(jnp.exp(logits - m), axis=1, keepdims=True)
    col = jax.lax.broadcasted_iota(jnp.int32, logits.shape, 1)

    l = logits
    w_cols = []
    i_cols = []
    for _ in range(K):
        cur = jnp.max(l, axis=1, keepdims=True)
        idx = jnp.min(
            jnp.where(l == cur, col, N_EXP), axis=1, keepdims=True
        )
        w_cols.append(jnp.exp(cur - m) / s)
        i_cols.append(idx)
        l = jnp.where(col == idx, -jnp.inf, l)

    out_w_ref[...] = jnp.concatenate(w_cols, axis=1)
    out_i_ref[...] = jnp.concatenate(i_cols, axis=1)


@jax.jit
def kernel(hidden_states, weight):
    grid = (N_TOK // BLOCK,)
    out_w, out_i = pl.pallas_call(
        _gate_kernel,
        grid=grid,
        in_specs=[
            pl.BlockSpec((BLOCK, N_EXP), lambda i: (i, 0)),
            pl.BlockSpec((N_EXP, N_EXP), lambda i: (0, 0)),
        ],
        out_specs=[
            pl.BlockSpec((BLOCK, K), lambda i: (i, 0)),
            pl.BlockSpec((BLOCK, K), lambda i: (i, 0)),
        ],
        out_shape=[
            jax.ShapeDtypeStruct((N_TOK, K), jnp.float32),
            jax.ShapeDtypeStruct((N_TOK, K), jnp.int32),
        ],
    )(hidden_states, weight)
    return (out_w, out_i)


# speedup vs baseline: 1.9656x; 1.9656x over previous
import jax
import jax.numpy as jnp
from jax.experimental import pallas as pl

N_TOK = 8192
N_EXP = 64
K = 8
BLOCK = 1024
NEG_INF = float("-inf")


def _gate_kernel(x_ref, w_ref, out_w_ref, out_i_ref):
    x = x_ref[...]
    w = w_ref[...]
    lt = jax.lax.dot_general(
        w, x, (((1,), (1,)), ((), ())), preferred_element_type=jnp.float32
    )
    eidx = jax.lax.broadcasted_iota(jnp.int32, lt.shape, 0)
    kiota = jax.lax.broadcasted_iota(jnp.int32, (K, BLOCK), 0)

    m = jnp.max(lt, axis=0, keepdims=True)
    s = jnp.sum(jnp.exp(lt - m), axis=0, keepdims=True)

    l = lt
    vals = jnp.zeros((K, BLOCK), jnp.float32)
    idxs = jnp.zeros((K, BLOCK), jnp.int32)
    for k in range(K):
        cur = jnp.max(l, axis=0, keepdims=True)
        idx = jnp.min(jnp.where(l == cur, eidx, N_EXP), axis=0, keepdims=True)
        vals = jnp.where(kiota == k, cur, vals)
        idxs = jnp.where(kiota == k, idx, idxs)
        if k + 1 < K:
            l = jnp.where(eidx == idx, NEG_INF, l)

    wts = jnp.exp(vals - m) / s
    out_w_ref[...] = wts.T
    out_i_ref[...] = idxs.T


@jax.jit
def kernel(hidden_states, weight):
    grid = (N_TOK // BLOCK,)
    out_w, out_i = pl.pallas_call(
        _gate_kernel,
        grid=grid,
        in_specs=[
            pl.BlockSpec((BLOCK, N_EXP), lambda i: (i, 0)),
            pl.BlockSpec((N_EXP, N_EXP), lambda i: (0, 0)),
        ],
        out_specs=[
            pl.BlockSpec((BLOCK, K), lambda i: (i, 0)),
            pl.BlockSpec((BLOCK, K), lambda i: (i, 0)),
        ],
        out_shape=[
            jax.ShapeDtypeStruct((N_TOK, K), jnp.float32),
            jax.ShapeDtypeStruct((N_TOK, K), jnp.int32),
        ],
    )(hidden_states, weight)
    return (out_w, out_i)
